# baseline (speedup 1.0000x reference)
import functools

import jax
import jax.numpy as jnp
from jax import lax
from jax.experimental import pallas as pl
from jax.experimental.pallas import tpu as pltpu
from jax.experimental.pallas import tpu_sc as plsc

N_NODES = 10000
N_EDGES = 320000
NFEAT = 128
NHID = 128
NCLASS = 64
FH = NHID // 2

NCORES = 2
NSUB = 16
NTILES = NCORES * NSUB
NPAD = 10240
ROWS_PER_TILE = NPAD // NSUB
TRASH = 10100
BLK = 128
CB = 40
NRING = 8

NB1 = 160
EPT1 = NB1 * BLK
EPAD = NSUB * EPT1
NB2 = 80

DEGW = 16
_SC_PARAMS = None


def _mesh():
    return plsc.VectorSubcoreMesh(core_axis_name="c", subcore_axis_name="s",
                                  num_cores=NCORES, num_subcores=NSUB)


def _sc_params():
    return pltpu.CompilerParams(use_tc_tiling_on_sc=False)


@functools.cache
def _make_deg_kernel():
    @functools.partial(
        pl.kernel,
        out_type=jax.ShapeDtypeStruct((NCORES, NPAD, DEGW), jnp.float32),
        mesh=_mesh(),
        scratch_types=[
            pltpu.VMEM((NB2, BLK), jnp.int32),
            pltpu.VMEM((BLK, DEGW), jnp.float32),
            pltpu.VMEM_SHARED((NPAD, DEGW), jnp.float32),
        ],
        compiler_params=_sc_params(),
    )
    def _deg_kernel(dst_hbm, ones_hbm, zeros_hbm, out_hbm,
                    dst_v, ones_v, acc_sh):
        cid = lax.axis_index("c")
        sid = lax.axis_index("s")
        tid = cid * NSUB + sid
        r0 = sid * ROWS_PER_TILE
        pltpu.sync_copy(zeros_hbm, acc_sh.at[pl.ds(r0, ROWS_PER_TILE)])
        pltpu.sync_copy(ones_hbm, ones_v)
        pltpu.sync_copy(dst_hbm.at[tid], dst_v)
        plsc.subcore_barrier()

        @pl.loop(0, NB2)
        def _(j):
            pltpu.sync_copy(ones_v, acc_sh.at[dst_v.at[j]], add=True)

        plsc.subcore_barrier()
        pltpu.sync_copy(acc_sh.at[pl.ds(r0, ROWS_PER_TILE)],
                        out_hbm.at[cid].at[pl.ds(r0, ROWS_PER_TILE)])

    return _deg_kernel


def _edge_chunk(y_ref, src_v, dst_v, rows_v, acc_sh, gsems, ssems,
                nring=NRING):
    def gather(j, s):
        pltpu.async_copy(y_ref.at[src_v.at[j]], rows_v.at[s], gsems[s])

    def wait_gather(j, s):
        pltpu.make_async_copy(y_ref.at[src_v.at[j]], rows_v.at[s],
                              gsems[s]).wait()

    def scatter(j, s):
        pltpu.async_copy(rows_v.at[s], acc_sh.at[dst_v.at[j]], ssems[s],
                         add=True)

    def wait_scatter(j, s):
        pltpu.make_async_copy(rows_v.at[s], acc_sh.at[dst_v.at[j]],
                              ssems[s]).wait()

    for b in range(nring - 1):
        gather(b, b)

    @pl.loop(0, CB, step=nring)
    def _(j):
        for b in range(nring):
            t = j + b
            s = b
            sp = (b + nring - 1) % nring
            wait_gather(t, s)
            scatter(t, s)

            @pl.when(t + nring - 1 < CB)
            def _():
                @pl.when(t - 1 >= 0)
                def _():
                    wait_scatter(t - 1, sp)

                gather(t + nring - 1, sp)

    for b in range(nring):
        t = CB - nring + b
        wait_scatter(t, t % nring)


NRING1 = 2


@functools.cache
def _make_edge1_kernel():
    @functools.partial(
        pl.kernel,
        out_type=jax.ShapeDtypeStruct((NCORES, NPAD, NHID), jnp.float32),
        mesh=_mesh(),
        scratch_types=[
            pltpu.VMEM((CB, BLK), jnp.int32),
            pltpu.VMEM((CB, BLK), jnp.int32),
            pltpu.VMEM((NRING1, BLK, NHID), jnp.float32),
            pltpu.VMEM_SHARED((NPAD, NHID), jnp.float32),
        ]
        + [pltpu.SemaphoreType.DMA] * (2 * NRING1),
        compiler_params=_sc_params(),
    )
    def _edge1(y_hbm, src_hbm, dst_hbm, zeros_hbm, out_hbm,
               src_v, dst_v, rows_v, acc_sh, *sems):
        gsems, ssems = sems[:NRING1], sems[NRING1:]
        cid = lax.axis_index("c")
        sid = lax.axis_index("s")
        tid = cid * NSUB + sid
        r0 = sid * ROWS_PER_TILE
        pltpu.sync_copy(zeros_hbm, acc_sh.at[pl.ds(r0, ROWS_PER_TILE)])
        plsc.subcore_barrier()

        for c in range(NB2 // CB):
            pltpu.sync_copy(src_hbm.at[tid].at[pl.ds(c * CB, CB)], src_v)
            pltpu.sync_copy(dst_hbm.at[tid].at[pl.ds(c * CB, CB)], dst_v)
            _edge_chunk(y_hbm, src_v, dst_v, rows_v, acc_sh, gsems, ssems,
                        nring=NRING1)

        plsc.subcore_barrier()
        pltpu.sync_copy(acc_sh.at[pl.ds(r0, ROWS_PER_TILE)],
                        out_hbm.at[cid].at[pl.ds(r0, ROWS_PER_TILE)])

    return _edge1


@functools.cache
def _make_edge2_kernel():
    @functools.partial(
        pl.kernel,
        out_type=jax.ShapeDtypeStruct((NCORES, NPAD, NCLASS), jnp.float32),
        mesh=_mesh(),
        scratch_types=[
            pltpu.VMEM((CB, BLK), jnp.int32),
            pltpu.VMEM((CB, BLK), jnp.int32),
            pltpu.VMEM((NRING, BLK, NCLASS), jnp.float32),
            pltpu.VMEM_SHARED((NPAD, NCLASS), jnp.float32),
        ]
        + [pltpu.SemaphoreType.DMA] * (2 * NRING),
        compiler_params=_sc_params(),
    )
    def _edge2(y_hbm, src_hbm, dst_hbm, zeros_hbm, out_hbm,
               src_v, dst_v, rows_v, acc_sh, *sems):
        gsems, ssems = sems[:NRING], sems[NRING:]
        cid = lax.axis_index("c")
        sid = lax.axis_index("s")
        tid = cid * NSUB + sid
        r0 = sid * ROWS_PER_TILE
        pltpu.sync_copy(zeros_hbm, acc_sh.at[pl.ds(r0, ROWS_PER_TILE)])
        plsc.subcore_barrier()
        for c in range(NB2 // CB):
            pltpu.sync_copy(src_hbm.at[tid].at[pl.ds(c * CB, CB)], src_v)
            pltpu.sync_copy(dst_hbm.at[tid].at[pl.ds(c * CB, CB)], dst_v)
            _edge_chunk(y_hbm, src_v, dst_v, rows_v, acc_sh, gsems, ssems)

        plsc.subcore_barrier()
        pltpu.sync_copy(acc_sh.at[pl.ds(r0, ROWS_PER_TILE)],
                        out_hbm.at[cid].at[pl.ds(r0, ROWS_PER_TILE)])

    return _edge2


def _dinv_from_deg(deg_ref):
    deg = deg_ref[0, :, 0:1] + deg_ref[1, :, 0:1] + 1.0
    return lax.rsqrt(deg)


def _y1_body(x_ref, w_ref, deg_ref, o_ref):
    dinv = _dinv_from_deg(deg_ref)
    o_ref[...] = jnp.dot(x_ref[...], w_ref[...],
                         preferred_element_type=jnp.float32) * dinv


def _y2_body(acc_ref, y1_ref, w_ref, deg_ref, o_ref):
    dinv = _dinv_from_deg(deg_ref)
    h = jnp.maximum((acc_ref[0] + acc_ref[1] + y1_ref[...]) * dinv, 0.0)
    o_ref[...] = jnp.dot(h, w_ref[...],
                         preferred_element_type=jnp.float32) * dinv


def _out_body(acc_ref, y2_ref, deg_ref, o_ref):
    dinv = _dinv_from_deg(deg_ref)
    o_ref[...] = jnp.maximum(
        (acc_ref[0] + acc_ref[1] + y2_ref[...]) * dinv, 0.0)


_y1_call = pl.pallas_call(
    _y1_body,
    out_shape=jax.ShapeDtypeStruct((NPAD, NHID), jnp.float32),
)

_y2_call = pl.pallas_call(
    _y2_body,
    out_shape=jax.ShapeDtypeStruct((NPAD, NCLASS), jnp.float32),
)

_out_call = pl.pallas_call(
    _out_body,
    out_shape=jax.ShapeDtypeStruct((NPAD, NCLASS), jnp.float32),
)


def kernel(x, adj, W1, W3):
    src = adj[0].astype(jnp.int32)
    dst = adj[1].astype(jnp.int32)
    pad = EPAD - N_EDGES
    pad_idx = N_NODES + jnp.arange(pad, dtype=jnp.int32) % (NPAD - N_NODES)
    src = jnp.concatenate([src, pad_idx])
    dst = jnp.concatenate([dst, pad_idx])
    src2 = src.reshape(NTILES, NB2, BLK)
    dst2 = dst.reshape(NTILES, NB2, BLK)

    x_pad = jnp.zeros((NPAD, NFEAT), jnp.float32).at[:N_NODES].set(x)
    ones_blk = jnp.ones((BLK, DEGW), jnp.float32)
    zeros_w = jnp.zeros((ROWS_PER_TILE, DEGW), jnp.float32)
    zeros_h = jnp.zeros((ROWS_PER_TILE, NHID), jnp.float32)
    zeros_c = jnp.zeros((ROWS_PER_TILE, NCLASS), jnp.float32)

    deg2 = _make_deg_kernel()(dst2, ones_blk, zeros_w)
    y1 = _y1_call(x_pad, W1, deg2)
    acc1 = _make_edge1_kernel()(y1, src2, dst2, zeros_h)
    y2 = _y2_call(acc1, y1, W3, deg2)
    acc2 = _make_edge2_kernel()(y2, src2, dst2, zeros_c)
    out = _out_call(acc2, y2, deg2)
    return out[:N_NODES]


# speedup vs baseline: 1.0727x; 1.0727x over previous
import functools

import jax
import jax.numpy as jnp
from jax import lax
from jax.experimental import pallas as pl
from jax.experimental.pallas import tpu as pltpu
from jax.experimental.pallas import tpu_sc as plsc

N_NODES = 10000
N_EDGES = 320000
NFEAT = 128
NHID = 128
NCLASS = 64
FH = NHID // 2

NCORES = 2
NSUB = 16
NTILES = NCORES * NSUB
NPAD = 10240
ROWS_PER_TILE = NPAD // NSUB
BLK = 128
CB = 40
NRING = 8

NB1 = 160
EPT1 = NB1 * BLK
EPAD = NSUB * EPT1
NB2 = 80

DEGW = 16
_SC_PARAMS = None


def _mesh():
    return plsc.VectorSubcoreMesh(core_axis_name="c", subcore_axis_name="s",
                                  num_cores=NCORES, num_subcores=NSUB)


def _sc_params():
    return pltpu.CompilerParams(use_tc_tiling_on_sc=False)


EROWS = N_EDGES // BLK
DB = EROWS // NTILES
DREM = EROWS - DB * NTILES
NSD = 8


@functools.cache
def _make_deg_kernel():
    @functools.partial(
        pl.kernel,
        out_type=jax.ShapeDtypeStruct((NCORES, NPAD, DEGW), jnp.float32),
        mesh=_mesh(),
        scratch_types=[
            pltpu.VMEM((DB, BLK), jnp.int32),
            pltpu.VMEM((1, BLK), jnp.int32),
            pltpu.VMEM((BLK, DEGW), jnp.float32),
            pltpu.VMEM_SHARED((NPAD, DEGW), jnp.float32),
        ]
        + [pltpu.SemaphoreType.DMA] * NSD,
        compiler_params=_sc_params(),
    )
    def _deg_kernel(adj_hbm, ones_hbm, zeros_hbm, out_hbm,
                    dst_v, extra_v, ones_v, acc_sh, *dsems):
        cid = lax.axis_index("c")
        sid = lax.axis_index("s")
        tid = cid * NSUB + sid
        r0 = sid * ROWS_PER_TILE
        pltpu.sync_copy(zeros_hbm, acc_sh.at[pl.ds(r0, ROWS_PER_TILE)])
        pltpu.sync_copy(ones_hbm, ones_v)
        pltpu.sync_copy(adj_hbm.at[1].at[pl.ds(tid * DB, DB)], dst_v)
        plsc.subcore_barrier()

        def fire(j, s):
            pltpu.async_copy(ones_v, acc_sh.at[dst_v.at[j]], dsems[s],
                             add=True)

        def drain(j, s):
            pltpu.make_async_copy(ones_v, acc_sh.at[dst_v.at[j]],
                                  dsems[s]).wait()

        main = (DB // NSD) * NSD

        @pl.loop(0, main, step=NSD)
        def _(j):
            for b in range(NSD):
                fire(j + b, b)
            for b in range(NSD):
                drain(j + b, b)

        for b in range(DB - main):
            fire(main + b, b)
        for b in range(DB - main):
            drain(main + b, b)

        @pl.when(tid < DREM)
        def _():
            pltpu.sync_copy(adj_hbm.at[1].at[pl.ds(NTILES * DB + tid, 1)],
                            extra_v)
            pltpu.sync_copy(ones_v, acc_sh.at[extra_v.at[0]], add=True)

        plsc.subcore_barrier()
        pltpu.sync_copy(acc_sh.at[pl.ds(r0, ROWS_PER_TILE)],
                        out_hbm.at[cid].at[pl.ds(r0, ROWS_PER_TILE)])

    return _deg_kernel


def _edge_chunk(y_ref, src_v, dst_v, rows_v, acc_sh, gsems, ssems):
    def gather(j, s):
        pltpu.async_copy(y_ref.at[src_v.at[j]], rows_v.at[s], gsems[s])

    def wait_gather(j, s):
        pltpu.make_async_copy(y_ref.at[src_v.at[j]], rows_v.at[s],
                              gsems[s]).wait()

    def scatter(j, s):
        pltpu.async_copy(rows_v.at[s], acc_sh.at[dst_v.at[j]], ssems[s],
                         add=True)

    def wait_scatter(j, s):
        pltpu.make_async_copy(rows_v.at[s], acc_sh.at[dst_v.at[j]],
                              ssems[s]).wait()

    for b in range(NRING - 1):
        gather(b, b)

    @pl.loop(0, CB, step=NRING)
    def _(j):
        for b in range(NRING):
            t = j + b
            s = b
            sp = (b + NRING - 1) % NRING
            wait_gather(t, s)
            scatter(t, s)

            @pl.when(t + NRING - 1 < CB)
            def _():
                @pl.when(t - 1 >= 0)
                def _():
                    wait_scatter(t - 1, sp)

                gather(t + NRING - 1, sp)

    for b in range(NRING):
        t = CB - NRING + b
        wait_scatter(t, t % NRING)


@functools.cache
def _make_edge1_kernel():
    @functools.partial(
        pl.kernel,
        out_type=jax.ShapeDtypeStruct((NCORES, NPAD, FH), jnp.float32),
        mesh=_mesh(),
        scratch_types=[
            pltpu.VMEM((CB, BLK), jnp.int32),
            pltpu.VMEM((CB, BLK), jnp.int32),
            pltpu.VMEM((NRING, BLK, FH), jnp.float32),
            pltpu.VMEM_SHARED((NPAD, FH), jnp.float32),
        ]
        + [pltpu.SemaphoreType.DMA] * (2 * NRING),
        compiler_params=_sc_params(),
    )
    def _edge1(y_hbm, src_hbm, dst_hbm, zeros_hbm, out_hbm,
               src_v, dst_v, rows_v, acc_sh, *sems):
        gsems, ssems = sems[:NRING], sems[NRING:]
        cid = lax.axis_index("c")
        sid = lax.axis_index("s")
        r0 = sid * ROWS_PER_TILE
        pltpu.sync_copy(zeros_hbm, acc_sh.at[pl.ds(r0, ROWS_PER_TILE)])
        plsc.subcore_barrier()
        y_ref = y_hbm.at[cid]

        for c in range(NB1 // CB):
            pltpu.sync_copy(src_hbm.at[sid].at[pl.ds(c * CB, CB)], src_v)
            pltpu.sync_copy(dst_hbm.at[sid].at[pl.ds(c * CB, CB)], dst_v)
            _edge_chunk(y_ref, src_v, dst_v, rows_v, acc_sh, gsems, ssems)

        plsc.subcore_barrier()
        pltpu.sync_copy(acc_sh.at[pl.ds(r0, ROWS_PER_TILE)],
                        out_hbm.at[cid].at[pl.ds(r0, ROWS_PER_TILE)])

    return _edge1


@functools.cache
def _make_edge2_kernel():
    @functools.partial(
        pl.kernel,
        out_type=jax.ShapeDtypeStruct((NCORES, NPAD, NCLASS), jnp.float32),
        mesh=_mesh(),
        scratch_types=[
            pltpu.VMEM((CB, BLK), jnp.int32),
            pltpu.VMEM((CB, BLK), jnp.int32),
            pltpu.VMEM((NRING, BLK, NCLASS), jnp.float32),
            pltpu.VMEM_SHARED((NPAD, NCLASS), jnp.float32),
        ]
        + [pltpu.SemaphoreType.DMA] * (2 * NRING),
        compiler_params=_sc_params(),
    )
    def _edge2(y_hbm, src_hbm, dst_hbm, zeros_hbm, out_hbm,
               src_v, dst_v, rows_v, acc_sh, *sems):
        gsems, ssems = sems[:NRING], sems[NRING:]
        cid = lax.axis_index("c")
        sid = lax.axis_index("s")
        tid = cid * NSUB + sid
        r0 = sid * ROWS_PER_TILE
        pltpu.sync_copy(zeros_hbm, acc_sh.at[pl.ds(r0, ROWS_PER_TILE)])
        plsc.subcore_barrier()
        for c in range(NB2 // CB):
            pltpu.sync_copy(src_hbm.at[tid].at[pl.ds(c * CB, CB)], src_v)
            pltpu.sync_copy(dst_hbm.at[tid].at[pl.ds(c * CB, CB)], dst_v)
            _edge_chunk(y_hbm, src_v, dst_v, rows_v, acc_sh, gsems, ssems)

        plsc.subcore_barrier()
        pltpu.sync_copy(acc_sh.at[pl.ds(r0, ROWS_PER_TILE)],
                        out_hbm.at[cid].at[pl.ds(r0, ROWS_PER_TILE)])

    return _edge2


EPADROWS = EPAD // BLK


def _pad_body(adj_ref, osrc_ref, odst_ref):
    pr = lax.broadcasted_iota(jnp.int32, (EPADROWS - EROWS, BLK), 0)
    pc = lax.broadcasted_iota(jnp.int32, (EPADROWS - EROWS, BLK), 1)
    padv = N_NODES + (pr * BLK + pc) % (NPAD - N_NODES)
    osrc_ref[:EROWS] = adj_ref[0]
    osrc_ref[EROWS:] = padv
    odst_ref[:EROWS] = adj_ref[1]
    odst_ref[EROWS:] = padv


_pad_call = pl.pallas_call(
    _pad_body,
    out_shape=[jax.ShapeDtypeStruct((EPADROWS, BLK), jnp.int32),
               jax.ShapeDtypeStruct((EPADROWS, BLK), jnp.int32)],
)


def _dinv_from_deg(deg_ref):
    deg = deg_ref[0, :, 0:1] + deg_ref[1, :, 0:1] + 1.0
    return lax.rsqrt(deg)


def _y1_body(x_ref, w_ref, deg_ref, o_ref):
    dinv = _dinv_from_deg(deg_ref)
    y = jnp.dot(x_ref[...], w_ref[...],
                preferred_element_type=jnp.float32) * dinv[:N_NODES]
    zpad = jnp.zeros((NPAD - N_NODES, FH), jnp.float32)
    o_ref[0, :N_NODES] = y[:, :FH]
    o_ref[0, N_NODES:] = zpad
    o_ref[1, :N_NODES] = y[:, FH:]
    o_ref[1, N_NODES:] = zpad


def _y2_body(acc_ref, y1_ref, w_ref, deg_ref, o_ref):
    dinv = _dinv_from_deg(deg_ref)
    hs = jnp.maximum((acc_ref[...] + y1_ref[...]) * dinv[None], 0.0)
    h = jnp.concatenate([hs[0], hs[1]], axis=1)
    o_ref[...] = jnp.dot(h, w_ref[...],
                         preferred_element_type=jnp.float32) * dinv


def _out_body(acc_ref, y2_ref, deg_ref, o_ref):
    dinv = _dinv_from_deg(deg_ref)
    o_ref[...] = jnp.maximum(
        (acc_ref[0, :N_NODES] + acc_ref[1, :N_NODES] + y2_ref[:N_NODES])
        * dinv[:N_NODES], 0.0)


_y1_call = pl.pallas_call(
    _y1_body,
    out_shape=jax.ShapeDtypeStruct((NCORES, NPAD, FH), jnp.float32),
)

_y2_call = pl.pallas_call(
    _y2_body,
    out_shape=jax.ShapeDtypeStruct((NPAD, NCLASS), jnp.float32),
)

_out_call = pl.pallas_call(
    _out_body,
    out_shape=jax.ShapeDtypeStruct((N_NODES, NCLASS), jnp.float32),
)


def kernel(x, adj, W1, W3):
    adj3 = adj.astype(jnp.int32).reshape(2, EROWS, BLK)

    ones_blk = jnp.ones((BLK, DEGW), jnp.float32)
    zeros_w = jnp.zeros((ROWS_PER_TILE, DEGW), jnp.float32)
    zeros_h = jnp.zeros((ROWS_PER_TILE, FH), jnp.float32)
    zeros_c = jnp.zeros((ROWS_PER_TILE, NCLASS), jnp.float32)

    deg2 = _make_deg_kernel()(adj3, ones_blk, zeros_w)
    src_p, dst_p = _pad_call(adj3)
    src1 = src_p.reshape(NSUB, NB1, BLK)
    dst1 = dst_p.reshape(NSUB, NB1, BLK)
    src2 = src_p.reshape(NTILES, NB2, BLK)
    dst2 = dst_p.reshape(NTILES, NB2, BLK)

    y1 = _y1_call(x, W1, deg2)
    acc1 = _make_edge1_kernel()(y1, src1, dst1, zeros_h)
    y2 = _y2_call(acc1, y1, W3, deg2)
    acc2 = _make_edge2_kernel()(y2, src2, dst2, zeros_c)
    return _out_call(acc2, y2, deg2)
